# baseline (speedup 1.0000x reference)
import functools

import jax
import jax.numpy as jnp
from jax import lax
from jax.experimental import pallas as pl
from jax.experimental.pallas import tpu as pltpu
from jax.experimental.pallas import tpu_sc as plsc

NUM_EMBEDDINGS = 1000000
D = 64
LN_EPS = 1e-5
ROWS = 4096
COLS = 200
N = ROWS * COLS

NUM_CORES = 2
NUM_SUBCORES = 16
NW = NUM_CORES * NUM_SUBCORES
PER_W = N // NW
C = 128
CHUNKS = PER_W // C
GROUPS = C // 16
INV_D = 1.0 / D


def _rsqrt(x):
    i = plsc.bitcast(x, jnp.int32)
    i = jnp.int32(0x5F3759DF) - (i >> 1)
    y = plsc.bitcast(i, jnp.float32)
    for _ in range(3):
        y = y * (1.5 - 0.5 * x * y * y)
    return y


def _make_kernel():
    mesh = plsc.VectorSubcoreMesh(core_axis_name="c", subcore_axis_name="s")

    @functools.partial(
        pl.kernel,
        mesh=mesh,
        out_type=jax.ShapeDtypeStruct((N, D), jnp.float32),
        scratch_types=[
            pltpu.VMEM((C,), jnp.int32),
            pltpu.VMEM((C, D), jnp.float32),
            pltpu.VMEM((D,), jnp.float32),
            pltpu.VMEM((D,), jnp.float32),
            pltpu.SemaphoreType.DMA,
        ],
        compiler_params=pltpu.CompilerParams(
            needs_layout_passes=False, use_tc_tiling_on_sc=False
        ),
    )
    def k(idx_hbm, table_hbm, gamma_hbm, beta_hbm, out_hbm,
          idx_v, rows_v, g_v, b_v, sem):
        wid = lax.axis_index("s") * NUM_CORES + lax.axis_index("c")
        base = wid * PER_W
        pltpu.sync_copy(gamma_hbm, g_v)
        pltpu.sync_copy(beta_hbm, b_v)
        lanes16 = jnp.arange(16, dtype=jnp.int32)

        def chunk_body(ci, carry):
            off = base + ci * C
            pltpu.sync_copy(idx_hbm.at[pl.ds(off, C)], idx_v)
            for j in range(C // 16):
                sl = pl.ds(j * 16, 16)
                idx_v[sl] = jnp.minimum(idx_v[sl], NUM_EMBEDDINGS - 1)
            pltpu.async_copy(table_hbm.at[idx_v], rows_v, sem).wait()

            def group_body(g, gcarry):
                rows = g * 16 + lanes16
                s = jnp.zeros((16,), jnp.float32)
                s2 = jnp.zeros((16,), jnp.float32)
                for c in range(D):
                    col = jnp.full((16,), c, jnp.int32)
                    x = plsc.load_gather(rows_v, [rows, col])
                    s = s + x
                    s2 = s2 + x * x
                mean = s * INV_D
                var = jnp.maximum(s2 * INV_D - mean * mean, 0.0) + LN_EPS
                rstd = _rsqrt(var)
                mrs = mean * rstd
                for c in range(D):
                    col = jnp.full((16,), c, jnp.int32)
                    x = plsc.load_gather(rows_v, [rows, col])
                    gc = plsc.load_gather(g_v, [col])
                    bc = plsc.load_gather(b_v, [col])
                    out = x * (rstd * gc) + (bc - mrs * gc)
                    plsc.store_scatter(rows_v, [rows, col], out)
                return gcarry

            lax.fori_loop(0, GROUPS, group_body, 0)
            pltpu.sync_copy(rows_v, out_hbm.at[pl.ds(off, C)])
            return carry

        lax.fori_loop(0, CHUNKS, chunk_body, 0)

    return k


_kernel_fn = _make_kernel()


def kernel(emb_indices, table, gamma, beta):
    flat_idx = emb_indices.reshape(-1)
    out = _kernel_fn(flat_idx, table, gamma, beta)
    return out.reshape(*emb_indices.shape, D)


# speedup vs baseline: 1.0935x; 1.0935x over previous
import functools

import jax
import jax.numpy as jnp
from jax import lax
from jax.experimental import pallas as pl
from jax.experimental.pallas import tpu as pltpu
from jax.experimental.pallas import tpu_sc as plsc

NUM_EMBEDDINGS = 1000000
D = 64
LN_EPS = 1e-5
N = 4096 * 200

NUM_CORES = 2
NUM_SUBCORES = 16
NW = NUM_CORES * NUM_SUBCORES
PER_W = N // NW
C = 256
CHUNKS = PER_W // C
GROUPS = C // 16
G_SPLIT = C // 128
INV_D = 1.0 / D


def _rsqrt(x):
    i = plsc.bitcast(x, jnp.int32)
    i = jnp.int32(0x5F3759DF) - (i >> 1)
    y = plsc.bitcast(i, jnp.float32)
    for _ in range(3):
        y = y * (1.5 - 0.5 * x * y * y)
    return y


def _make_kernel():
    mesh = plsc.VectorSubcoreMesh(core_axis_name="c", subcore_axis_name="s")

    @functools.partial(
        pl.kernel,
        mesh=mesh,
        out_type=jax.ShapeDtypeStruct((N, D), jnp.float32),
        scratch_types=[
            pltpu.VMEM((PER_W,), jnp.int32),
            pltpu.VMEM((C, D), jnp.float32),
            pltpu.VMEM((C, D), jnp.float32),
            pltpu.VMEM((C, D), jnp.float32),
            pltpu.VMEM((C, D), jnp.float32),
            pltpu.VMEM((D,), jnp.float32),
            pltpu.VMEM((D,), jnp.float32),
            pltpu.VMEM((D, 16), jnp.float32),
            pltpu.VMEM((D, 16), jnp.float32),
            pltpu.SemaphoreType.DMA,
            pltpu.SemaphoreType.DMA,
            pltpu.SemaphoreType.DMA,
            pltpu.SemaphoreType.DMA,
        ],
        compiler_params=pltpu.CompilerParams(
            needs_layout_passes=False, use_tc_tiling_on_sc=False
        ),
    )
    def k(idx_hbm, table_hbm, gamma_hbm, beta_hbm, out_hbm,
          idx_all, rows0, rows1, ob0, ob1, g_v, b_v, gsp, bsp,
          gsem0, gsem1, osem0, osem1):
        wid = lax.axis_index("s") * NUM_CORES + lax.axis_index("c")
        base = wid * PER_W
        pltpu.sync_copy(idx_hbm.at[pl.ds(base, PER_W)], idx_all)
        pltpu.sync_copy(gamma_hbm, g_v)
        pltpu.sync_copy(beta_hbm, b_v)

        for c in range(D):
            col = jnp.full((16,), c, jnp.int32)
            gsp[c] = plsc.load_gather(g_v, [col])
            bsp[c] = plsc.load_gather(b_v, [col])

        lanes16 = jnp.arange(16, dtype=jnp.int32)
        rows_bufs = (rows0, rows1)
        out_bufs = (ob0, ob1)
        gsems = (gsem0, gsem1)
        osems = (osem0, osem1)

        def clamp_and_fire(ci, b):
            for j in range(C // 16):
                sl = pl.ds(ci * C + j * 16, 16)
                idx_all[sl] = jnp.minimum(idx_all[sl], NUM_EMBEDDINGS - 1)
            for h in range(G_SPLIT):
                src = table_hbm.at[idx_all.at[pl.ds(ci * C + h * 128, 128)]]
                pltpu.async_copy(src, rows_bufs[b].at[pl.ds(h * 128, 128)],
                                 gsems[b])

        def wait_gather(b):
            for h in range(G_SPLIT):
                pltpu.make_async_copy(
                    table_hbm.at[idx_all.at[pl.ds(h * 128, 128)]],
                    rows_bufs[b].at[pl.ds(h * 128, 128)],
                    gsems[b],
                ).wait()

        def fire_out(ci, b):
            pltpu.async_copy(out_bufs[b],
                             out_hbm.at[pl.ds(base + ci * C, C)], osems[b])

        def wait_out(b):
            pltpu.make_async_copy(out_bufs[b], out_hbm.at[pl.ds(base, C)],
                                  osems[b]).wait()

        def compute(b):
            rows_ref = rows_bufs[b]
            ob_ref = out_bufs[b]

            def group_body(g, gcarry):
                rows = g * 16 + lanes16
                acc = [jnp.zeros((16,), jnp.float32) for _ in range(4)]
                acc2 = [jnp.zeros((16,), jnp.float32) for _ in range(4)]
                for c in range(D):
                    col = jnp.full((16,), c, jnp.int32)
                    x = plsc.load_gather(rows_ref, [rows, col])
                    acc[c & 3] = acc[c & 3] + x
                    acc2[c & 3] = acc2[c & 3] + x * x
                s = (acc[0] + acc[1]) + (acc[2] + acc[3])
                s2 = (acc2[0] + acc2[1]) + (acc2[2] + acc2[3])
                mean = s * INV_D
                var = jnp.maximum(s2 * INV_D - mean * mean, 0.0) + LN_EPS
                rstd = _rsqrt(var)
                mrs = mean * rstd
                for c in range(D):
                    col = jnp.full((16,), c, jnp.int32)
                    x = plsc.load_gather(rows_ref, [rows, col])
                    gc = gsp[c]
                    bc = bsp[c]
                    out = x * (rstd * gc) + (bc - mrs * gc)
                    plsc.store_scatter(ob_ref, [rows, col], out)
                return gcarry

            lax.fori_loop(0, GROUPS, group_body, 0)

        clamp_and_fire(0, 0)

        def pair_body(cp, carry):
            for b in (0, 1):
                ci = cp * 2 + b
                wait_gather(b)
                pl.when(ci + 1 < CHUNKS)(
                    functools.partial(clamp_and_fire, ci + 1, 1 - b))
                pl.when(ci >= 2)(functools.partial(wait_out, b))
                compute(b)
                fire_out(ci, b)
            return carry

        lax.fori_loop(0, CHUNKS // 2, pair_body, 0)
        wait_out(0)
        wait_out(1)

    return k


_kernel_fn = _make_kernel()


def kernel(emb_indices, table, gamma, beta):
    flat_idx = emb_indices.reshape(-1)
    out = _kernel_fn(flat_idx, table, gamma, beta)
    return out.reshape(*emb_indices.shape, D)
